# baseline (speedup 1.0000x reference)
import functools

import jax
import jax.numpy as jnp
from jax.experimental import pallas as pl
from jax.experimental.pallas import tpu as pltpu

D_MODEL = 768
NUM_EXPERTS = 64
TOP_K = 2


def _router_kernel(x_ref, w_ref, idx_ref, wt_ref):
    logits = jax.lax.dot_general(
        w_ref[...], x_ref[...],
        dimension_numbers=(((1,), (1,)), ((), ())),
        preferred_element_type=jnp.float32,
    )
    row = jax.lax.broadcasted_iota(jnp.int32, logits.shape, 0)
    m1 = jnp.max(logits, axis=0, keepdims=True)
    i1 = jnp.min(jnp.where(logits == m1, row, NUM_EXPERTS), axis=0, keepdims=True)
    masked = jnp.where(row == i1, -jnp.inf, logits)
    m2 = jnp.max(masked, axis=0, keepdims=True)
    i2 = jnp.min(jnp.where(masked == m2, row, NUM_EXPERTS), axis=0, keepdims=True)
    w1 = 1.0 / (1.0 + jnp.exp(m2 - m1))
    w2 = 1.0 - w1
    idx_ref[...] = jnp.concatenate([i1, i2], axis=0)
    wt_ref[...] = jnp.concatenate([w1, w2], axis=0)


@functools.partial(jax.jit, static_argnames=("block_tokens",))
def _route(x2d, W, block_tokens):
    n_tokens = x2d.shape[0]
    grid = (n_tokens // block_tokens,)
    idx_t, wts_t = pl.pallas_call(
        _router_kernel,
        grid=grid,
        in_specs=[
            pl.BlockSpec((block_tokens, D_MODEL), lambda i: (i, 0)),
            pl.BlockSpec((NUM_EXPERTS, D_MODEL), lambda i: (0, 0)),
        ],
        out_specs=[
            pl.BlockSpec((TOP_K, block_tokens), lambda i: (0, i)),
            pl.BlockSpec((TOP_K, block_tokens), lambda i: (0, i)),
        ],
        out_shape=[
            jax.ShapeDtypeStruct((TOP_K, n_tokens), jnp.int32),
            jax.ShapeDtypeStruct((TOP_K, n_tokens), jnp.float32),
        ],
        compiler_params=pltpu.CompilerParams(
            dimension_semantics=("parallel",),
        ),
    )(x2d, W)
    return idx_t, wts_t


def kernel(x, W):
    b, s, d = x.shape
    x2d = x.reshape(b * s, d)
    idx_t, wts_t = _route(x2d, W, 2048)
    idx = idx_t.T.reshape(b, s, TOP_K)
    wts = wts_t.T.reshape(b, s, TOP_K)
    return idx, wts


# speedup vs baseline: 1.0265x; 1.0265x over previous
import functools

import jax
import jax.numpy as jnp
from jax.experimental import pallas as pl
from jax.experimental.pallas import tpu as pltpu

D_MODEL = 768
D_HALF = D_MODEL // 2
NUM_EXPERTS = 64
TOP_K = 2


def _router_kernel(x1_ref, x2_ref, w_ref, idx_ref, wt_ref):
    dn = (((1,), (1,)), ((), ()))
    logits = jax.lax.dot_general(
        w_ref[:, :D_HALF], x1_ref[...], dn, preferred_element_type=jnp.float32
    ) + jax.lax.dot_general(
        w_ref[:, D_HALF:], x2_ref[...], dn, preferred_element_type=jnp.float32
    )
    row = jax.lax.broadcasted_iota(jnp.int32, logits.shape, 0)
    m1 = jnp.max(logits, axis=0, keepdims=True)
    i1 = jnp.min(jnp.where(logits == m1, row, NUM_EXPERTS), axis=0, keepdims=True)
    masked = jnp.where(row == i1, -jnp.inf, logits)
    m2 = jnp.max(masked, axis=0, keepdims=True)
    i2 = jnp.min(jnp.where(masked == m2, row, NUM_EXPERTS), axis=0, keepdims=True)
    w1 = 1.0 / (1.0 + jnp.exp(m2 - m1))
    w2 = 1.0 - w1
    idx_ref[...] = jnp.concatenate([i1, i2], axis=0)
    wt_ref[...] = jnp.concatenate([w1, w2], axis=0)


@functools.partial(jax.jit, static_argnames=("block_tokens",))
def _route(x2d, W, block_tokens):
    n_tokens = x2d.shape[0]
    grid = (n_tokens // block_tokens,)
    idx_t, wts_t = pl.pallas_call(
        _router_kernel,
        grid=grid,
        in_specs=[
            pl.BlockSpec((block_tokens, D_HALF), lambda i: (i, 0)),
            pl.BlockSpec((block_tokens, D_HALF), lambda i: (i, 1)),
            pl.BlockSpec((NUM_EXPERTS, D_MODEL), lambda i: (0, 0)),
        ],
        out_specs=[
            pl.BlockSpec((TOP_K, block_tokens), lambda i: (0, i)),
            pl.BlockSpec((TOP_K, block_tokens), lambda i: (0, i)),
        ],
        out_shape=[
            jax.ShapeDtypeStruct((TOP_K, n_tokens), jnp.int32),
            jax.ShapeDtypeStruct((TOP_K, n_tokens), jnp.float32),
        ],
        compiler_params=pltpu.CompilerParams(
            dimension_semantics=("parallel",),
        ),
    )(x2d, x2d, W)
    return idx_t, wts_t


def kernel(x, W):
    b, s, d = x.shape
    x2d = x.reshape(b * s, d)
    idx_t, wts_t = _route(x2d, W, 4096)
    idx = idx_t.T.reshape(b, s, TOP_K)
    wts = wts_t.T.reshape(b, s, TOP_K)
    return idx, wts


# speedup vs baseline: 1.0375x; 1.0108x over previous
import functools

import jax
import jax.numpy as jnp
from jax.experimental import pallas as pl
from jax.experimental.pallas import tpu as pltpu

D_MODEL = 768
NUM_EXPERTS = 64
TOP_K = 2


def _router_kernel(x_ref, w_ref, idx_ref, wt_ref):
    logits = jax.lax.dot_general(
        w_ref[...], x_ref[...],
        dimension_numbers=(((1,), (1,)), ((), ())),
        preferred_element_type=jnp.float32,
    )
    row = jax.lax.broadcasted_iota(jnp.int32, logits.shape, 0).astype(jnp.float32)
    big = jnp.float32(NUM_EXPERTS)
    m1 = jnp.max(logits, axis=0, keepdims=True)
    i1 = jnp.min(jnp.where(logits == m1, row, big), axis=0, keepdims=True)
    masked = jnp.where(row == i1, -jnp.inf, logits)
    m2 = jnp.max(masked, axis=0, keepdims=True)
    i2 = jnp.min(jnp.where(masked == m2, row, big), axis=0, keepdims=True)
    w1 = 1.0 / (1.0 + jnp.exp(m2 - m1))
    w2 = 1.0 - w1
    idx_ref[...] = jnp.concatenate([i1, i2], axis=0).astype(jnp.int32)
    wt_ref[...] = jnp.concatenate([w1, w2], axis=0)


@functools.partial(jax.jit, static_argnames=("block_tokens",))
def _route(x2d, W, block_tokens):
    n_tokens = x2d.shape[0]
    grid = (n_tokens // block_tokens,)
    idx_t, wts_t = pl.pallas_call(
        _router_kernel,
        grid=grid,
        in_specs=[
            pl.BlockSpec((block_tokens, D_MODEL), lambda i: (i, 0)),
            pl.BlockSpec((NUM_EXPERTS, D_MODEL), lambda i: (0, 0)),
        ],
        out_specs=[
            pl.BlockSpec((TOP_K, block_tokens), lambda i: (0, i)),
            pl.BlockSpec((TOP_K, block_tokens), lambda i: (0, i)),
        ],
        out_shape=[
            jax.ShapeDtypeStruct((TOP_K, n_tokens), jnp.int32),
            jax.ShapeDtypeStruct((TOP_K, n_tokens), jnp.float32),
        ],
        compiler_params=pltpu.CompilerParams(
            dimension_semantics=("parallel",),
        ),
    )(x2d, W)
    return idx_t, wts_t


def kernel(x, W):
    b, s, d = x.shape
    x2d = x.reshape(b * s, d)
    idx_t, wts_t = _route(x2d, W, 4096)
    idx = idx_t.T.reshape(b, s, TOP_K)
    wts = wts_t.T.reshape(b, s, TOP_K)
    return idx, wts


# speedup vs baseline: 1.0568x; 1.0186x over previous
import functools

import jax
import jax.numpy as jnp
from jax.experimental import pallas as pl
from jax.experimental.pallas import tpu as pltpu

D_MODEL = 768
NUM_EXPERTS = 64
TOP_K = 2


def _router_kernel(x_ref, w_ref, idx_ref, wt_ref):
    logits = jax.lax.dot_general(
        w_ref[...], x_ref[...],
        dimension_numbers=(((1,), (1,)), ((), ())),
        preferred_element_type=jnp.float32,
    )
    row = jax.lax.broadcasted_iota(jnp.int32, logits.shape, 0)
    big = NUM_EXPERTS
    m1 = jnp.max(logits, axis=0, keepdims=True)
    i1 = jnp.min(jnp.where(logits == m1, row, big), axis=0, keepdims=True)
    masked = jnp.where(row == i1, -jnp.inf, logits)
    m2 = jnp.max(masked, axis=0, keepdims=True)
    i2 = jnp.min(jnp.where(masked == m2, row, big), axis=0, keepdims=True)
    w1 = 1.0 / (1.0 + jnp.exp(m2 - m1))
    w2 = 1.0 - w1
    idx_ref[...] = jnp.concatenate([i1, i2], axis=0)
    wt_ref[...] = jnp.concatenate([w1, w2], axis=0)


@functools.partial(jax.jit, static_argnames=("block_tokens",))
def _route(x2d, W, block_tokens):
    n_tokens = x2d.shape[0]
    grid = (n_tokens // block_tokens,)
    idx_t, wts_t = pl.pallas_call(
        _router_kernel,
        grid=grid,
        in_specs=[
            pl.BlockSpec((block_tokens, D_MODEL), lambda i: (i, 0)),
            pl.BlockSpec((NUM_EXPERTS, D_MODEL), lambda i: (0, 0)),
        ],
        out_specs=[
            pl.BlockSpec((TOP_K, block_tokens), lambda i: (0, i)),
            pl.BlockSpec((TOP_K, block_tokens), lambda i: (0, i)),
        ],
        out_shape=[
            jax.ShapeDtypeStruct((TOP_K, n_tokens), jnp.int32),
            jax.ShapeDtypeStruct((TOP_K, n_tokens), jnp.float32),
        ],
        compiler_params=pltpu.CompilerParams(
            dimension_semantics=("parallel",),
        ),
    )(x2d, W)
    return idx_t, wts_t


def kernel(x, W):
    b, s, d = x.shape
    x2d = x.reshape(b * s, d)
    idx_t, wts_t = _route(x2d, W, 4096)
    idx = idx_t.T.reshape(b, s, TOP_K)
    wts = wts_t.T.reshape(b, s, TOP_K)
    return idx, wts
